# baseline (speedup 1.0000x reference)
import math

import numpy as np
import jax
import jax.numpy as jnp
from jax import lax
from jax.experimental import pallas as pl
from jax.experimental.pallas import tpu as pltpu
from jax.experimental.pallas import tpu_sc as plsc

VOCAB_ROWS = 1000000
D = 64
B0 = 4096
S = 200
B_TOTAL = B0 * S
NC, NS, L = 2, 16, 16
NW = NC * NS
CHUNK = 128
BH = B0 // CHUNK
TOT_CHUNKS = B_TOTAL // CHUNK
N_CHUNKS = TOT_CHUNKS // NW
NBUF = 2
SCALE = math.sqrt(D)
TBLK = 512

_IOTA = np.arange(L, dtype=np.int32)


def _body(x_hbm, table_hbm, out_hbm, idx_v, in_v, t_v, gsem, ssem):
    wid = lax.axis_index("s") * NC + lax.axis_index("c")
    base_chunk = wid * N_CHUNKS
    pltpu.sync_copy(x_hbm.at[pl.ds(base_chunk, N_CHUNKS)], idx_v)

    iota = lax.iota(jnp.int32, L)
    one = jnp.full((L,), 1, jnp.int32)
    bsplat = [jnp.full((L,), b, jnp.int32) for b in range(NBUF)]

    def start_gather(t, b):
        pltpu.async_copy(table_hbm.at[idx_v.at[t]], in_v.at[b], gsem.at[b])

    def wait_gather(b):
        pltpu.make_async_copy(
            table_hbm.at[idx_v.at[0]], in_v.at[b], gsem.at[b]
        ).wait()

    def start_store(t, b):
        c = base_chunk + t
        s = c // BH
        bh = lax.rem(c, BH)
        pltpu.async_copy(t_v.at[b], out_hbm.at[s, :, bh], ssem.at[b])

    def wait_store(b):
        pltpu.make_async_copy(
            t_v.at[b], out_hbm.at[0, :, 0], ssem.at[b]
        ).wait()

    def scale_transpose(t, b):
        @plsc.parallel_loop(0, CHUNK // L, 1)
        def jg_loop(jg):
            j0 = jg * L
            jvec = iota + j0

            @plsc.parallel_loop(0, L, 1, unroll=2)
            def r_loop(r):
                rotv = lax.bitwise_and(iota + r, jnp.full((L,), L - 1, jnp.int32))
                rrow = lax.shift_right_logical(rotv, 3)
                rdl = lax.bitwise_and(rotv, jnp.full((L,), 7, jnp.int32))
                for db in range(D // L):
                    vals = plsc.load_gather(
                        in_v, [bsplat[b], jvec, (db * L) + rotv]
                    ) * SCALE
                    plsc.store_scatter(
                        t_v, [bsplat[b], rrow + 2 * db, rdl, jvec], vals
                    )

    for b in range(NBUF):
        start_gather(b, b)

    for b in range(NBUF):
        wait_gather(b)
        scale_transpose(b, b)
        start_store(b, b)
        start_gather(b + NBUF, b)

    @pl.loop(NBUF, N_CHUNKS - NBUF, step=NBUF)
    def round_loop(t0):
        for b in range(NBUF):
            wait_store(b)
            wait_gather(b)
            scale_transpose(t0 + b, b)
            start_store(t0 + b, b)
            start_gather(t0 + b + NBUF, b)

    for b in range(NBUF):
        t = N_CHUNKS - NBUF + b
        wait_store(b)
        wait_gather(b)
        scale_transpose(t, b)
        start_store(t, b)

    for b in range(NBUF):
        wait_store(b)


def _tc_body(tt_ref, o_ref):
    o_ref[:, :D] = jnp.transpose(tt_ref[:, :], (1, 0))
    o_ref[:, D:] = jnp.zeros((TBLK, CHUNK - D), jnp.float32)


@jax.jit
def _transpose_pad(tt):
    grid = (VOCAB_ROWS + TBLK - 1) // TBLK
    return pl.pallas_call(
        _tc_body,
        grid=(grid,),
        in_specs=[pl.BlockSpec((D, TBLK), lambda i: (0, i))],
        out_specs=pl.BlockSpec((TBLK, CHUNK), lambda i: (i, 0)),
        out_shape=jax.ShapeDtypeStruct((VOCAB_ROWS, CHUNK), jnp.float32),
    )(tt)


@jax.jit
def _embed(x2d, tablep):
    mesh = plsc.VectorSubcoreMesh(
        core_axis_name="c", subcore_axis_name="s", num_cores=NC, num_subcores=NS
    )
    run = pl.kernel(
        _body,
        out_type=jax.ShapeDtypeStruct((S, D // 8, BH, 8, CHUNK), jnp.float32),
        mesh=mesh,
        scratch_types=[
            pltpu.VMEM((N_CHUNKS, CHUNK), jnp.int32),
            pltpu.VMEM((NBUF, CHUNK, CHUNK), jnp.float32),
            pltpu.VMEM((NBUF, D // 8, 8, CHUNK), jnp.float32),
            pltpu.SemaphoreType.DMA((NBUF,)),
            pltpu.SemaphoreType.DMA((NBUF,)),
        ],
        compiler_params=pltpu.CompilerParams(
            use_tc_tiling_on_sc=True,
            needs_layout_passes=False,
            skip_device_barrier=True,
        ),
    )
    return run(x2d, tablep)


def kernel(x, pretrained_vector):
    x2d = x.T.reshape(TOT_CHUNKS, CHUNK).astype(jnp.int32)
    tablep = _transpose_pad(pretrained_vector.T)
    out4 = _embed(x2d, tablep)
    out = out4.transpose(2, 4, 0, 1, 3).reshape(B0, S, D)
    return out


# speedup vs baseline: 1.8458x; 1.8458x over previous
import math

import numpy as np
import jax
import jax.numpy as jnp
from jax import lax
from jax.experimental import pallas as pl
from jax.experimental.pallas import tpu as pltpu
from jax.experimental.pallas import tpu_sc as plsc

VOCAB_ROWS = 1000000
D = 64
B0 = 4096
S = 200
B_TOTAL = B0 * S
NC, NS, L = 2, 16, 16
NW = NC * NS
CHUNK = 128
BH = B0 // CHUNK
TOT_CHUNKS = B_TOTAL // CHUNK
N_CHUNKS = TOT_CHUNKS // NW
NBUF = 2
SCALE = math.sqrt(D)

_IOTA = np.arange(L, dtype=np.int32)


def _body(x_hbm, table_hbm, out_hbm, idx_v, in_v, t_v, gsem, ssem):
    wid = lax.axis_index("s") * NC + lax.axis_index("c")
    base_chunk = wid * N_CHUNKS
    pltpu.sync_copy(x_hbm.at[pl.ds(base_chunk, N_CHUNKS)], idx_v)

    iota = lax.iota(jnp.int32, L)
    one = jnp.full((L,), 1, jnp.int32)
    bsplat = [jnp.full((L,), b, jnp.int32) for b in range(NBUF)]

    def start_gather(t, b):
        pltpu.async_copy(table_hbm.at[idx_v.at[t]], in_v.at[b], gsem.at[b])

    def wait_gather(b):
        pltpu.make_async_copy(
            table_hbm.at[idx_v.at[0]], in_v.at[b], gsem.at[b]
        ).wait()

    def start_store(t, b):
        c = base_chunk + t
        s = c // BH
        bh = lax.rem(c, BH)
        pltpu.async_copy(t_v.at[b], out_hbm.at[s, :, bh], ssem.at[b])

    def wait_store(b):
        pltpu.make_async_copy(
            t_v.at[b], out_hbm.at[0, :, 0], ssem.at[b]
        ).wait()

    def scale_transpose(t, b):
        @plsc.parallel_loop(0, CHUNK // L, 1)
        def jg_loop(jg):
            j0 = jg * L
            jvec = iota + j0

            @plsc.parallel_loop(0, L, 1, unroll=2)
            def r_loop(r):
                rotv = lax.bitwise_and(iota + r, jnp.full((L,), L - 1, jnp.int32))
                rrow = lax.shift_right_logical(rotv, 3)
                rdl = lax.bitwise_and(rotv, jnp.full((L,), 7, jnp.int32))
                for db in range(D // L):
                    vals = plsc.load_gather(
                        in_v, [bsplat[b], jvec, (db * L) + rotv]
                    ) * SCALE
                    plsc.store_scatter(
                        t_v, [bsplat[b], rrow + 2 * db, rdl, jvec], vals
                    )

    for b in range(NBUF):
        start_gather(b, b)

    for b in range(NBUF):
        wait_gather(b)
        scale_transpose(b, b)
        start_store(b, b)
        start_gather(b + NBUF, b)

    @pl.loop(NBUF, N_CHUNKS - NBUF, step=NBUF)
    def round_loop(t0):
        for b in range(NBUF):
            wait_store(b)
            wait_gather(b)
            scale_transpose(t0 + b, b)
            start_store(t0 + b, b)
            start_gather(t0 + b + NBUF, b)

    for b in range(NBUF):
        t = N_CHUNKS - NBUF + b
        wait_store(b)
        wait_gather(b)
        scale_transpose(t, b)
        start_store(t, b)

    for b in range(NBUF):
        wait_store(b)


@jax.jit
def _embed(x2d, tablep):
    mesh = plsc.VectorSubcoreMesh(
        core_axis_name="c", subcore_axis_name="s", num_cores=NC, num_subcores=NS
    )
    run = pl.kernel(
        _body,
        out_type=jax.ShapeDtypeStruct((S, D // 8, BH, 8, CHUNK), jnp.float32),
        mesh=mesh,
        scratch_types=[
            pltpu.VMEM((N_CHUNKS, CHUNK), jnp.int32),
            pltpu.VMEM((NBUF, CHUNK, CHUNK), jnp.float32),
            pltpu.VMEM((NBUF, D // 8, 8, CHUNK), jnp.float32),
            pltpu.SemaphoreType.DMA((NBUF,)),
            pltpu.SemaphoreType.DMA((NBUF,)),
        ],
        compiler_params=pltpu.CompilerParams(
            use_tc_tiling_on_sc=True,
            needs_layout_passes=False,
            skip_device_barrier=True,
        ),
    )
    return run(x2d, tablep)


def kernel(x, pretrained_vector):
    x2d = x.T.reshape(TOT_CHUNKS, CHUNK).astype(jnp.int32)
    tablep = jnp.pad(pretrained_vector, ((0, 0), (0, CHUNK - D)))
    out4 = _embed(x2d, tablep)
    out = out4.transpose(2, 4, 0, 1, 3).reshape(B0, S, D)
    return out


# speedup vs baseline: 1.8486x; 1.0015x over previous
import math

import jax
import jax.numpy as jnp
from jax import lax
from jax.experimental import pallas as pl
from jax.experimental.pallas import tpu as pltpu
from jax.experimental.pallas import tpu_sc as plsc

VOCAB_ROWS = 1000000
D = 64
B0 = 4096
S = 200
B_TOTAL = B0 * S
NC, NS, L = 2, 16, 16
NW = NC * NS
CHUNK = 128
BH = B0 // CHUNK
TOT_CHUNKS = B_TOTAL // CHUNK
N_CHUNKS = TOT_CHUNKS // NW
NBUF = 2
SCALE = math.sqrt(D)

def _body(x_hbm, table_hbm, out_hbm, idx_v, in_v, t_v, gsem, ssem):
    wid = lax.axis_index("s") * NC + lax.axis_index("c")
    base_chunk = wid * N_CHUNKS
    pltpu.sync_copy(x_hbm.at[pl.ds(base_chunk, N_CHUNKS)], idx_v)

    iota = lax.iota(jnp.int32, L)
    bsplat = [jnp.full((L,), b, jnp.int32) for b in range(NBUF)]

    def start_gather(t, b):
        pltpu.async_copy(table_hbm.at[idx_v.at[t]], in_v.at[b], gsem.at[b])

    def wait_gather(b):
        pltpu.make_async_copy(
            table_hbm.at[idx_v.at[0]], in_v.at[b], gsem.at[b]
        ).wait()

    def start_store(t, b):
        c = base_chunk + t
        s = c // BH
        bh = lax.rem(c, BH)
        pltpu.async_copy(t_v.at[b], out_hbm.at[s, :, bh], ssem.at[b])

    def wait_store(b):
        pltpu.make_async_copy(
            t_v.at[b], out_hbm.at[0, :, 0], ssem.at[b]
        ).wait()

    def scale_transpose(t, b):
        @plsc.parallel_loop(0, CHUNK // L, 1)
        def jg_loop(jg):
            j0 = jg * L
            jvec = iota + j0

            @plsc.parallel_loop(0, L, 1, unroll=2)
            def r_loop(r):
                rotv = lax.bitwise_and(iota + r, jnp.full((L,), L - 1, jnp.int32))
                rrow = lax.shift_right_logical(rotv, 3)
                rdl = lax.bitwise_and(rotv, jnp.full((L,), 7, jnp.int32))
                for db in range(D // L):
                    vals = plsc.load_gather(
                        in_v, [bsplat[b], jvec, (db * L) + rotv]
                    ) * SCALE
                    plsc.store_scatter(
                        t_v, [bsplat[b], rrow + 2 * db, rdl, jvec], vals
                    )

    for b in range(NBUF):
        start_gather(b, b)

    for b in range(NBUF):
        wait_gather(b)
        scale_transpose(b, b)
        start_store(b, b)
        start_gather(b + NBUF, b)

    @pl.loop(NBUF, N_CHUNKS - NBUF, step=NBUF)
    def round_loop(t0):
        for b in range(NBUF):
            wait_store(b)
            wait_gather(b)
            scale_transpose(t0 + b, b)
            start_store(t0 + b, b)
            start_gather(t0 + b + NBUF, b)

    for b in range(NBUF):
        t = N_CHUNKS - NBUF + b
        wait_store(b)
        wait_gather(b)
        scale_transpose(t, b)
        start_store(t, b)

    for b in range(NBUF):
        wait_store(b)


@jax.jit
def _embed(x2d, tablep):
    mesh = plsc.VectorSubcoreMesh(
        core_axis_name="c", subcore_axis_name="s", num_cores=NC, num_subcores=NS
    )
    run = pl.kernel(
        _body,
        out_type=jax.ShapeDtypeStruct((S, D // 8, BH, 8, CHUNK), jnp.float32),
        mesh=mesh,
        scratch_types=[
            pltpu.VMEM((N_CHUNKS, CHUNK), jnp.int32),
            pltpu.VMEM((NBUF, CHUNK, CHUNK), jnp.float32),
            pltpu.VMEM((NBUF, D // 8, 8, CHUNK), jnp.float32),
            pltpu.SemaphoreType.DMA((NBUF,)),
            pltpu.SemaphoreType.DMA((NBUF,)),
        ],
        compiler_params=pltpu.CompilerParams(
            use_tc_tiling_on_sc=True,
            needs_layout_passes=False,
            skip_device_barrier=True,
        ),
    )
    return run(x2d, tablep)


def kernel(x, pretrained_vector):
    x2d = x.T.reshape(TOT_CHUNKS, CHUNK).astype(jnp.int32)
    tablep = jnp.pad(pretrained_vector, ((0, 0), (0, CHUNK - D)))
    out4 = _embed(x2d, tablep)
    out = out4.transpose(2, 4, 0, 1, 3).reshape(B0, S, D)
    return out
